# baseline (speedup 1.0000x reference)
import dataclasses
import functools

import jax
import jax.numpy as jnp
from jax import lax
from jax.experimental import pallas as pl
from jax.experimental.pallas import tpu as pltpu
from jax.experimental.pallas import tpu_sc as plsc

_NCU = 1
_NS = 16
_C = 80
_NB = 10
_ZR = 128
_NPAD = 10240


def _proj_body(x_ref, w_ref, b_ref, p_ref):
    p_ref[...] = jnp.dot(x_ref[...], w_ref[0], preferred_element_type=jnp.float32,
                         precision=lax.Precision.HIGHEST) + b_ref[0]


def _project(x, wst, bst, block_rows=2000):
    n, d = x.shape
    dout = wst.shape[2]
    nblocks = n // block_rows
    return pl.pallas_call(
        _proj_body,
        grid=(2, nblocks),
        in_specs=[
            pl.BlockSpec((block_rows, d), lambda j, i: (i, 0)),
            pl.BlockSpec((1, d, dout), lambda j, i: (j, 0, 0)),
            pl.BlockSpec((1, 1, dout), lambda j, i: (j, 0, 0)),
        ],
        out_specs=pl.BlockSpec((block_rows, dout),
                               lambda j, i, nb=nblocks: (j * nb + i, 0)),
        out_shape=jax.ShapeDtypeStruct((2 * n, dout), jnp.float32),
    )(x, wst, bst)


def _final_body(x_ref, agg_ref, w1_ref, w2_ref, b_ref, o_ref):
    z = jnp.dot(x_ref[...], w1_ref[...], preferred_element_type=jnp.float32,
                precision=lax.Precision.HIGHEST)
    z += jnp.dot(agg_ref[...], w2_ref[...], preferred_element_type=jnp.float32,
                 precision=lax.Precision.HIGHEST)
    o_ref[...] = jnp.maximum(z + b_ref[...], 0.0)


def _final(x, agg, w1t, w2t, b2d, block_rows=2000):
    n, d = x.shape
    dout = w2t.shape[1]
    return pl.pallas_call(
        _final_body,
        grid=(n // block_rows,),
        in_specs=[
            pl.BlockSpec((block_rows, d), lambda i: (i, 0)),
            pl.BlockSpec((block_rows, dout), lambda i: (i, 0)),
            pl.BlockSpec((d, dout), lambda i: (0, 0)),
            pl.BlockSpec((dout, dout), lambda i: (0, 0)),
            pl.BlockSpec((1, dout), lambda i: (0, 0)),
        ],
        out_specs=pl.BlockSpec((block_rows, dout), lambda i: (i, 0)),
        out_shape=jax.ShapeDtypeStruct((n, dout), jnp.float32),
    )(x, agg, w1t, w2t, b2d)


def _edge_agg(p, gidx, sidx):
    d = p.shape[1]
    nblk = gidx.shape[1]
    rows_tile = _NPAD // _NS
    nz = rows_tile // _ZR

    mesh = plsc.VectorSubcoreMesh(core_axis_name="c", subcore_axis_name="s",
                                  num_cores=_NCU)
    cp = pltpu.CompilerParams()
    if "needs_layout_passes" in pltpu.CompilerParams.__dataclass_fields__:
        cp = dataclasses.replace(cp, needs_layout_passes=False)

    @functools.partial(
        pl.kernel,
        out_type=jax.ShapeDtypeStruct((_NPAD, d), jnp.float32),
        mesh=mesh,
        compiler_params=cp,
        scratch_types=[
            pltpu.VMEM((_NB, _C), jnp.int32),
            pltpu.VMEM((_NB, _C), jnp.int32),
            pltpu.VMEM((_C, d), jnp.float32),
            pltpu.VMEM((_C, d), jnp.float32),
            pltpu.VMEM((_C, d), jnp.float32),
            pltpu.VMEM((_C, d), jnp.float32),
            pltpu.VMEM_SHARED((_NPAD, d), jnp.float32),
            pltpu.SemaphoreType.DMA,
            pltpu.SemaphoreType.DMA,
        ],
    )
    def k(p_hbm, c4_hbm, r4_hbm, out_hbm,
          cidx_ib, ridx_ib, ubuf0, ubuf1, vbuf0, vbuf1, agg_sh, sem_g, sem_s):
        s = lax.axis_index("s")
        ub = (ubuf0, ubuf1)
        vb = (vbuf0, vbuf1)

        @pl.loop(0, _C)
        def _(i):
            for g in range(d // 16):
                ubuf0[i, pl.ds(g * 16, 16)] = jnp.zeros((16,), jnp.float32)

        rbase = s * rows_tile
        for z in range(rows_tile // _C):
            pltpu.sync_copy(ubuf0, agg_sh.at[pl.ds(rbase + z * _C, _C)])
        plsc.subcore_barrier()

        @pl.loop(0, nblk)
        def _(k_):
            pltpu.sync_copy(c4_hbm.at[s, k_], cidx_ib)
            pltpu.sync_copy(r4_hbm.at[s, k_], ridx_ib)

            pltpu.async_copy(p_hbm.at[ridx_ib.at[0]], ubuf0, sem_g)
            pltpu.async_copy(p_hbm.at[cidx_ib.at[0]], vbuf0, sem_g)

            @pl.loop(0, _NB, step=2)
            def _(tt):
                for b in (0, 1):
                    t = tt + b
                    pltpu.make_async_copy(p_hbm.at[ridx_ib.at[t]], ub[b],
                                          sem_g).wait()
                    pltpu.make_async_copy(p_hbm.at[cidx_ib.at[t]], vb[b],
                                          sem_g).wait()

                    def _wait_prev_scatter():
                        pltpu.make_async_copy(
                            vb[1 - b], agg_sh.at[ridx_ib.at[0]], sem_s).wait()
                    if b == 1:
                        _wait_prev_scatter()
                    else:
                        pl.when(tt >= 1)(_wait_prev_scatter)

                    def _issue_next_gathers():
                        pltpu.async_copy(p_hbm.at[ridx_ib.at[t + 1]], ub[1 - b],
                                         sem_g)
                        pltpu.async_copy(p_hbm.at[cidx_ib.at[t + 1]], vb[1 - b],
                                         sem_g)
                    if b == 0:
                        _issue_next_gathers()
                    else:
                        pl.when(tt < _NB - 2)(_issue_next_gathers)

                    @pl.loop(0, _C)
                    def _(r):
                        for g in range(d // 16):
                            sl = pl.ds(g * 16, 16)
                            vb[b][r, sl] = jnp.maximum(
                                ub[b][r, sl] + vb[b][r, sl], 0.0)

                    pltpu.async_copy(vb[b], agg_sh.at[ridx_ib.at[t]], sem_s,
                                     add=True)

            pltpu.make_async_copy(vbuf1, agg_sh.at[ridx_ib.at[0]], sem_s).wait()

        plsc.subcore_barrier()

        for z in range(nz):
            r0 = rbase + z * _ZR
            pltpu.sync_copy(agg_sh.at[pl.ds(r0, _ZR)], out_hbm.at[pl.ds(r0, _ZR)])

    return k(p, gidx, sidx)


def kernel(x, edge_index, W, b):
    n, d = x.shape
    dout = W.shape[0]
    w1t = jnp.transpose(W[:, :d])
    w2t = jnp.transpose(W[:, d:])
    b2d = b.reshape(1, dout)
    wst = jnp.stack([w1t, w2t])
    bst = jnp.concatenate([jnp.zeros((1, dout), jnp.float32), b2d],
                          axis=0).reshape(2, 1, dout)

    rows = edge_index[0]
    cols = edge_index[1]
    e = rows.shape[0]
    nw = _NCU * _NS
    nblk = e // (nw * _NB * _C)
    rows4 = rows.reshape(nw, nblk, _NB, _C)
    colsn4 = cols.reshape(nw, nblk, _NB, _C) + n

    p = _project(x, wst, bst)
    agg = _edge_agg(p, colsn4, rows4)
    return _final(x, agg, w1t, w2t, b2d)
